# baseline (speedup 1.0000x reference)
import functools

import jax
import jax.numpy as jnp
from jax import lax
from jax.experimental import pallas as pl
from jax.experimental.pallas import tpu as pltpu
from jax.experimental.pallas import tpu_sc as plsc

USER_NUM = 100000
ITEM_NUM = 50000
D = 64
BPR_W = 0.7
CAUSAL_W = 0.3

NCHUNK = 4
CW = 16
NS = 16
E_BLK = 1024
BLKS_PER_TILE = 62
NNZ_PAD = NS * BLKS_PER_TILE * E_BLK
GRPS = E_BLK // 16


def _gcn_sc(users4, items4, noise4, rows_p, cols_p, vals_p):
    f32 = jnp.float32
    u_sds = jax.ShapeDtypeStruct((NCHUNK, USER_NUM, CW), f32)
    i_sds = jax.ShapeDtypeStruct((NCHUNK, ITEM_NUM, CW), f32)
    mesh = plsc.VectorSubcoreMesh(core_axis_name="c", subcore_axis_name="s")

    @functools.partial(
        pl.kernel,
        out_type=(u_sds, u_sds, i_sds, i_sds, u_sds, i_sds),
        mesh=mesh,
        compiler_params=pltpu.CompilerParams(use_tc_tiling_on_sc=False),
        scratch_types=[
            pltpu.VMEM_SHARED((USER_NUM, CW), f32),
            pltpu.VMEM((E_BLK,), jnp.int32),
            pltpu.VMEM((E_BLK,), jnp.int32),
            pltpu.VMEM((E_BLK,), f32),
            pltpu.VMEM((E_BLK, CW), f32),
            pltpu.SemaphoreType.DMA,
            pltpu.SemaphoreType.DMA,
        ],
    )
    def k(users_h, items_h, noise_h, rows_h, cols_h, vals_h,
          g1u_h, g2u_h, g1i_h, g2i_h, n1u_h, n2i_h,
          acc, gidx, sidx, valsv, gath, sem, semi):
        c = lax.axis_index("c")
        s = lax.axis_index("s")

        def spmm(src_h, gidx_h, sidx_h, init_h, out_h, n_out):
            rpt = n_out // NS
            r0 = s * rpt
            for kl in range(2):
                ck = 2 * c + kl
                pltpu.sync_copy(init_h.at[ck, pl.ds(r0, rpt)],
                                acc.at[pl.ds(r0, rpt)])
                plsc.subcore_barrier()

                def blk(b, carry):
                    e0 = (s * BLKS_PER_TILE + b) * E_BLK
                    d1 = pltpu.async_copy(gidx_h.at[pl.ds(e0, E_BLK)], gidx, semi)
                    d2 = pltpu.async_copy(sidx_h.at[pl.ds(e0, E_BLK)], sidx, semi)
                    d3 = pltpu.async_copy(vals_h.at[pl.ds(e0, E_BLK)], valsv, semi)
                    d1.wait(); d2.wait(); d3.wait()
                    pltpu.async_copy(src_h.at[ck].at[gidx], gath, sem).wait()

                    def grp(g, carry2):
                        vv = valsv[pl.ds(g * 16, 16)]
                        for j in range(16):
                            e = g * 16 + j
                            gath[e, :] = gath[e, :] * vv[j]
                        return carry2

                    lax.fori_loop(0, GRPS, grp, 0, unroll=False)
                    pltpu.sync_copy(gath, acc.at[sidx], add=True)
                    return carry

                lax.fori_loop(0, BLKS_PER_TILE, blk, 0, unroll=False)
                plsc.subcore_barrier()
                pltpu.sync_copy(acc.at[pl.ds(r0, rpt)],
                                out_h.at[ck, pl.ds(r0, rpt)])
                plsc.subcore_barrier()

        spmm(users_h, rows_h, cols_h, items_h, g1i_h, ITEM_NUM)
        spmm(items_h, cols_h, rows_h, users_h, g1u_h, USER_NUM)
        spmm(noise_h, cols_h, rows_h, g1u_h, n1u_h, USER_NUM)
        spmm(g1i_h, cols_h, rows_h, g1u_h, g2u_h, USER_NUM)
        spmm(g1u_h, rows_h, cols_h, g1i_h, g2i_h, ITEM_NUM)
        spmm(n1u_h, rows_h, cols_h, g1i_h, n2i_h, ITEM_NUM)

    return k(users4, items4, noise4, rows_p, cols_p, vals_p)


def _final_body(u_ref, t_ref, o_ref):
    o_ref[...] = jnp.sum(u_ref[...] * t_ref[...], axis=-1)


def _final_dots(u_big, t_big):
    n = u_big.shape[0]
    blk = 2048
    return pl.pallas_call(
        _final_body,
        out_shape=jax.ShapeDtypeStruct((n,), jnp.float32),
        grid=(n // blk,),
        in_specs=[
            pl.BlockSpec((blk, u_big.shape[1]), lambda i: (i, 0)),
            pl.BlockSpec((blk, u_big.shape[1]), lambda i: (i, 0)),
        ],
        out_specs=pl.BlockSpec((blk,), lambda i: (i,)),
    )(u_big, t_big)


def _chunked(x):
    n = x.shape[0]
    return x.reshape(n, NCHUNK, CW).transpose(1, 0, 2)


def _gat(t4, idx):
    g = t4[:, idx, :]
    return g.transpose(1, 0, 2).reshape(idx.shape[0], D)


def kernel(u_batch, i_batch, j_batch, embed_user, embed_item, causal_user,
           causal_item, noise_item, ui_rows, ui_cols, ui_vals):
    users4 = _chunked(BPR_W * embed_user + CAUSAL_W * causal_user)
    items4 = _chunked(BPR_W * embed_item + CAUSAL_W * causal_item)
    noise4 = _chunked(noise_item)

    pad = NNZ_PAD - ui_rows.shape[0]
    rows_p = jnp.concatenate((ui_rows.astype(jnp.int32),
                              jnp.zeros((pad,), jnp.int32)))
    cols_p = jnp.concatenate((ui_cols.astype(jnp.int32),
                              jnp.zeros((pad,), jnp.int32)))
    vals_p = jnp.concatenate((ui_vals, jnp.zeros((pad,), jnp.float32)))

    g1u4, g2u4, g1i4, g2i4, n1u4, n2i4p = _gcn_sc(
        users4, items4, noise4, rows_p, cols_p, vals_p)

    noise_based4 = items4 + noise4
    n1i4 = g1i4 + noise4
    n2i4 = n2i4p + noise4

    B = u_batch.shape[0]
    half = B // 2
    u_b = u_batch.astype(jnp.int32)

    u_big = jnp.concatenate(
        (_gat(users4, u_b), _gat(g1u4, u_b), _gat(g2u4, u_b)), axis=-1)

    i_lo, i_hi = i_batch[:half], i_batch[half:]
    pos_t = jnp.concatenate((
        jnp.concatenate((_gat(items4, i_lo), _gat(g1i4, i_lo), _gat(g2i4, i_lo)), axis=-1),
        jnp.concatenate((_gat(noise_based4, i_hi), _gat(n1i4, i_hi), _gat(n2i4, i_hi)), axis=-1),
    ), axis=0)

    j_lo, j_hi = j_batch[:half], j_batch[half:]
    neg_t = jnp.concatenate((
        jnp.concatenate((_gat(noise_based4, j_hi), _gat(n1i4, j_hi), _gat(n2i4, j_hi)), axis=-1),
        jnp.concatenate((_gat(items4, j_lo), _gat(g1i4, j_lo), _gat(g2i4, j_lo)), axis=-1),
    ), axis=0)

    u2 = jnp.concatenate((u_big, u_big), axis=0)
    t2 = jnp.concatenate((pos_t, neg_t), axis=0)
    preds = _final_dots(u2, t2)
    return preds.reshape(2, B)


# speedup vs baseline: 1.4866x; 1.4866x over previous
import functools

import jax
import jax.numpy as jnp
from jax import lax
from jax.experimental import pallas as pl
from jax.experimental.pallas import tpu as pltpu
from jax.experimental.pallas import tpu_sc as plsc

USER_NUM = 100000
ITEM_NUM = 50000
D = 64
BPR_W = 0.7
CAUSAL_W = 0.3

NCHUNK = 4
CW = 16
NS = 16
E_BLK = 1024
BLKS_PER_TILE = 62
NNZ_PAD = NS * BLKS_PER_TILE * E_BLK
GRPS = E_BLK // 16


def _gcn_sc(users4, items4, noise4, rows_p, cols_p, vals_p,
            u_batch, i_batch, j_batch):
    f32 = jnp.float32
    B = u_batch.shape[0]
    bpt = B // NS
    half_tiles = (B // 2) // bpt
    u_sds = jax.ShapeDtypeStruct((NCHUNK, USER_NUM, CW), f32)
    i_sds = jax.ShapeDtypeStruct((NCHUNK, ITEM_NUM, CW), f32)
    b_sds = jax.ShapeDtypeStruct((B, 3 * NCHUNK, CW), f32)
    mesh = plsc.VectorSubcoreMesh(core_axis_name="c", subcore_axis_name="s")

    @functools.partial(
        pl.kernel,
        out_type=(u_sds, u_sds, i_sds, i_sds, u_sds, i_sds,
                  b_sds, b_sds, b_sds),
        mesh=mesh,
        compiler_params=pltpu.CompilerParams(use_tc_tiling_on_sc=False),
        scratch_types=[
            pltpu.VMEM_SHARED((USER_NUM, CW), f32),
            pltpu.VMEM((E_BLK,), jnp.int32),
            pltpu.VMEM((E_BLK,), jnp.int32),
            pltpu.VMEM((E_BLK,), f32),
            pltpu.VMEM((E_BLK, CW), f32),
            pltpu.VMEM((E_BLK // 2, CW), f32),
            pltpu.SemaphoreType.DMA,
            pltpu.SemaphoreType.DMA,
        ],
    )
    def k(users_h, items_h, noise_h, rows_h, cols_h, vals_h,
          ub_idx_h, ib_idx_h, jb_idx_h,
          g1u_h, g2u_h, g1i_h, g2i_h, n1u_h, n2i_h,
          ub_h, pt_h, nt_h,
          acc, gidx, sidx, valsv, gath, nbuf, sem, semi):
        c = lax.axis_index("c")
        s = lax.axis_index("s")

        def spmm(src_h, gidx_h, sidx_h, init_h, out_h, n_out):
            rpt = n_out // NS
            r0 = s * rpt
            for kl in range(2):
                ck = 2 * c + kl
                pltpu.sync_copy(init_h.at[ck, pl.ds(r0, rpt)],
                                acc.at[pl.ds(r0, rpt)])
                plsc.subcore_barrier()

                def blk(b, carry):
                    e0 = (s * BLKS_PER_TILE + b) * E_BLK
                    d1 = pltpu.async_copy(gidx_h.at[pl.ds(e0, E_BLK)], gidx, semi)
                    d2 = pltpu.async_copy(sidx_h.at[pl.ds(e0, E_BLK)], sidx, semi)
                    d3 = pltpu.async_copy(vals_h.at[pl.ds(e0, E_BLK)], valsv, semi)
                    d1.wait(); d2.wait(); d3.wait()
                    pltpu.async_copy(src_h.at[ck].at[gidx], gath, sem).wait()

                    def grp(g, carry2):
                        vv = valsv[pl.ds(g * 16, 16)]
                        for j in range(16):
                            e = g * 16 + j
                            gath[e, :] = gath[e, :] * vv[j]
                        return carry2

                    lax.fori_loop(0, GRPS, grp, 0, unroll=False)
                    pltpu.sync_copy(gath, acc.at[sidx], add=True)
                    return carry

                lax.fori_loop(0, BLKS_PER_TILE, blk, 0, unroll=False)
                plsc.subcore_barrier()
                pltpu.sync_copy(acc.at[pl.ds(r0, rpt)],
                                out_h.at[ck, pl.ds(r0, rpt)])
                plsc.subcore_barrier()

        spmm(users_h, rows_h, cols_h, items_h, g1i_h, ITEM_NUM)
        spmm(items_h, cols_h, rows_h, users_h, g1u_h, USER_NUM)
        spmm(noise_h, cols_h, rows_h, g1u_h, n1u_h, USER_NUM)
        spmm(g1i_h, cols_h, rows_h, g1u_h, g2u_h, USER_NUM)
        spmm(g1u_h, rows_h, cols_h, g1i_h, g2i_h, ITEM_NUM)
        spmm(n1u_h, rows_h, cols_h, g1i_h, n2i_h, ITEM_NUM)

        r0b = s * bpt

        def plain_gather(idx_h, off, tabs, out_h):
            pltpu.sync_copy(idx_h.at[pl.ds(off, bpt)], gidx.at[pl.ds(0, bpt)])
            for kl in range(2):
                ck = 2 * c + kl
                for t, tab in enumerate(tabs):
                    pltpu.async_copy(tab.at[ck].at[gidx.at[pl.ds(0, bpt)]],
                                     gath.at[pl.ds(0, bpt)], sem).wait()
                    pltpu.sync_copy(gath.at[pl.ds(0, bpt)],
                                    out_h.at[pl.ds(r0b, bpt), t * NCHUNK + ck])

        def noisy_gather(idx_h, off, tabs, out_h):
            hb = bpt // 2
            pltpu.sync_copy(idx_h.at[pl.ds(off, bpt)], gidx.at[pl.ds(0, bpt)])
            for kl in range(2):
                ck = 2 * c + kl
                for t, tab in enumerate(tabs):
                    pltpu.async_copy(tab.at[ck].at[gidx.at[pl.ds(0, bpt)]],
                                     gath.at[pl.ds(0, bpt)], sem).wait()
                    for sb in range(2):
                        pltpu.async_copy(
                            noise_h.at[ck].at[gidx.at[pl.ds(sb * hb, hb)]],
                            nbuf.at[pl.ds(0, hb)], sem).wait()

                        def addrow(r, carry):
                            e = sb * hb + r
                            gath[e, :] = gath[e, :] + nbuf[r, :]
                            return carry

                        lax.fori_loop(0, hb, addrow, 0, unroll=False)
                    pltpu.sync_copy(gath.at[pl.ds(0, bpt)],
                                    out_h.at[pl.ds(r0b, bpt), t * NCHUNK + ck])

        plain_gather(ub_idx_h, r0b, (users_h, g1u_h, g2u_h), ub_h)
        base_tabs = (items_h, g1i_h, g2i_h)
        noisy_tabs = (items_h, g1i_h, n2i_h)

        @pl.when(s < half_tiles)
        def _():
            plain_gather(ib_idx_h, r0b, base_tabs, pt_h)
            noisy_gather(jb_idx_h, r0b + B // 2, noisy_tabs, nt_h)

        @pl.when(s >= half_tiles)
        def _():
            noisy_gather(ib_idx_h, r0b, noisy_tabs, pt_h)
            plain_gather(jb_idx_h, r0b - B // 2, base_tabs, nt_h)

    return k(users4, items4, noise4, rows_p, cols_p, vals_p,
             u_batch, i_batch, j_batch)


def _final_body(u_ref, p_ref, n_ref, o_ref):
    u = u_ref[...]
    o_ref[0, :] = jnp.sum(u * p_ref[...], axis=-1)
    o_ref[1, :] = jnp.sum(u * n_ref[...], axis=-1)


def _final_dots(ub, pt, nt):
    n = ub.shape[0]
    blk = 2048
    return pl.pallas_call(
        _final_body,
        out_shape=jax.ShapeDtypeStruct((2, n), jnp.float32),
        grid=(n // blk,),
        in_specs=[
            pl.BlockSpec((blk, ub.shape[1]), lambda i: (i, 0)),
            pl.BlockSpec((blk, ub.shape[1]), lambda i: (i, 0)),
            pl.BlockSpec((blk, ub.shape[1]), lambda i: (i, 0)),
        ],
        out_specs=pl.BlockSpec((2, blk), lambda i: (0, i)),
    )(ub, pt, nt)


def _chunked(x):
    n = x.shape[0]
    return x.reshape(n, NCHUNK, CW).transpose(1, 0, 2)


def _gat(t4, idx):
    g = t4[:, idx, :]
    return g.transpose(1, 0, 2).reshape(idx.shape[0], D)


def kernel(u_batch, i_batch, j_batch, embed_user, embed_item, causal_user,
           causal_item, noise_item, ui_rows, ui_cols, ui_vals):
    users4 = _chunked(BPR_W * embed_user + CAUSAL_W * causal_user)
    items4 = _chunked(BPR_W * embed_item + CAUSAL_W * causal_item)
    noise4 = _chunked(noise_item)

    pad = NNZ_PAD - ui_rows.shape[0]
    rows_p = jnp.concatenate((ui_rows.astype(jnp.int32),
                              jnp.zeros((pad,), jnp.int32)))
    cols_p = jnp.concatenate((ui_cols.astype(jnp.int32),
                              jnp.zeros((pad,), jnp.int32)))
    vals_p = jnp.concatenate((ui_vals, jnp.zeros((pad,), jnp.float32)))

    B = u_batch.shape[0]
    outs = _gcn_sc(users4, items4, noise4, rows_p, cols_p, vals_p,
                   u_batch.astype(jnp.int32), i_batch.astype(jnp.int32),
                   j_batch.astype(jnp.int32))
    ub_h, pt_h, nt_h = outs[6], outs[7], outs[8]
    return _final_dots(ub_h.reshape(B, 3 * D), pt_h.reshape(B, 3 * D),
                       nt_h.reshape(B, 3 * D))


# speedup vs baseline: 2.5267x; 1.6996x over previous
import functools

import jax
import jax.numpy as jnp
from jax import lax
from jax.experimental import pallas as pl
from jax.experimental.pallas import tpu as pltpu
from jax.experimental.pallas import tpu_sc as plsc

USER_NUM = 100000
ITEM_NUM = 50000
D = 64
BPR_W = 0.7
CAUSAL_W = 0.3

NCHUNK = 4
CW = 16
NS = 16
E_BLK = 512
NB = 123
NNZ_PAD = NS * NB * E_BLK
GRPS = E_BLK // 16
SUB = 512


def _gcn_sc(users4, items4, noise4, rows_p, cols_p, vals_p,
            u_batch, i_batch, j_batch):
    f32 = jnp.float32
    B = u_batch.shape[0]
    bpt = B // NS
    half_tiles = (B // 2) // bpt
    u_sds = jax.ShapeDtypeStruct((NCHUNK, USER_NUM, CW), f32)
    i_sds = jax.ShapeDtypeStruct((NCHUNK, ITEM_NUM, CW), f32)
    b_sds = jax.ShapeDtypeStruct((B, 3 * NCHUNK, CW), f32)
    mesh = plsc.VectorSubcoreMesh(core_axis_name="c", subcore_axis_name="s")

    @functools.partial(
        pl.kernel,
        out_type=(u_sds, u_sds, i_sds, i_sds, u_sds, i_sds,
                  b_sds, b_sds, b_sds),
        mesh=mesh,
        compiler_params=pltpu.CompilerParams(use_tc_tiling_on_sc=False),
        scratch_types=[
            pltpu.VMEM_SHARED((USER_NUM, CW), f32),
            pltpu.VMEM((3, E_BLK, CW), f32),
            pltpu.VMEM((4, E_BLK), jnp.int32),
            pltpu.VMEM((4, E_BLK), jnp.int32),
            pltpu.VMEM((4, E_BLK), f32),
            pltpu.SemaphoreType.DMA,
            pltpu.SemaphoreType.DMA,
            pltpu.SemaphoreType.DMA,
        ],
    )
    def k(users_h, items_h, noise_h, rows_h, cols_h, vals_h,
          ub_idx_h, ib_idx_h, jb_idx_h,
          g1u_h, g2u_h, g1i_h, g2i_h, n1u_h, n2i_h,
          ub_h, pt_h, nt_h,
          acc, gath3, gidx4, sidx4, valsv4, semi, semg, semsc):
        c = lax.axis_index("c")
        s = lax.axis_index("s")

        def spmm(src_h, gidx_h, sidx_h, init_h, out_h, n_out):
            rpt = n_out // NS
            r0 = s * rpt
            for kl in range(2):
                ck = 2 * c + kl
                pltpu.sync_copy(init_h.at[ck, pl.ds(r0, rpt)],
                                acc.at[pl.ds(r0, rpt)])
                plsc.subcore_barrier()

                def issue_idx(b, r):
                    e0 = (s * NB + b) * E_BLK
                    pltpu.async_copy(gidx_h.at[pl.ds(e0, E_BLK)],
                                     gidx4.at[r], semi)
                    pltpu.async_copy(sidx_h.at[pl.ds(e0, E_BLK)],
                                     sidx4.at[r], semi)
                    pltpu.async_copy(vals_h.at[pl.ds(e0, E_BLK)],
                                     valsv4.at[r], semi)

                def wait_idx(b, r):
                    e0 = (s * NB + b) * E_BLK
                    pltpu.make_async_copy(gidx_h.at[pl.ds(e0, E_BLK)],
                                          gidx4.at[r], semi).wait()
                    pltpu.make_async_copy(sidx_h.at[pl.ds(e0, E_BLK)],
                                          sidx4.at[r], semi).wait()
                    pltpu.make_async_copy(vals_h.at[pl.ds(e0, E_BLK)],
                                          valsv4.at[r], semi).wait()

                def issue_gather(r, p):
                    pltpu.async_copy(src_h.at[ck].at[gidx4.at[r]],
                                     gath3.at[p], semg)

                def wait_gather(r, p):
                    pltpu.make_async_copy(src_h.at[ck].at[gidx4.at[r]],
                                          gath3.at[p], semg).wait()

                def issue_scatter(r, p):
                    pltpu.async_copy(gath3.at[p], acc.at[sidx4.at[r]],
                                     semsc, add=True)

                def wait_scatter(r, p):
                    pltpu.make_async_copy(gath3.at[p], acc.at[sidx4.at[r]],
                                          semsc).wait()

                def scale(r, p):
                    def grp(g, carry):
                        vv = valsv4[r, pl.ds(g * 16, 16)]
                        for j in range(16):
                            e = g * 16 + j
                            gath3[p, e, :] = gath3[p, e, :] * vv[j]
                        return carry

                    lax.fori_loop(0, GRPS, grp, 0, unroll=False)

                issue_idx(0, 0)

                def blk(b, carry):
                    p = lax.rem(b, 3)
                    r = lax.rem(b, 4)

                    @pl.when(b >= 3)
                    def _():
                        wait_scatter(lax.rem(b + 1, 4), lax.rem(b, 3))

                    @pl.when(b + 1 < NB)
                    def _():
                        issue_idx(b + 1, lax.rem(b + 1, 4))

                    wait_idx(b, r)
                    issue_gather(r, p)

                    @pl.when(b >= 1)
                    def _():
                        rm1 = lax.rem(b + 3, 4)
                        pm1 = lax.rem(b + 2, 3)
                        wait_gather(rm1, pm1)
                        scale(rm1, pm1)
                        issue_scatter(rm1, pm1)

                    return carry

                lax.fori_loop(0, NB, blk, 0, unroll=False)
                rL, pL = (NB - 1) % 4, (NB - 1) % 3
                wait_gather(rL, pL)
                scale(rL, pL)
                issue_scatter(rL, pL)
                wait_scatter((NB - 3) % 4, (NB - 3) % 3)
                wait_scatter((NB - 2) % 4, (NB - 2) % 3)
                wait_scatter(rL, pL)
                plsc.subcore_barrier()
                pltpu.sync_copy(acc.at[pl.ds(r0, rpt)],
                                out_h.at[ck, pl.ds(r0, rpt)])
                plsc.subcore_barrier()

        spmm(users_h, rows_h, cols_h, items_h, g1i_h, ITEM_NUM)
        spmm(items_h, cols_h, rows_h, users_h, g1u_h, USER_NUM)
        spmm(noise_h, cols_h, rows_h, g1u_h, n1u_h, USER_NUM)
        spmm(g1i_h, cols_h, rows_h, g1u_h, g2u_h, USER_NUM)
        spmm(g1u_h, rows_h, cols_h, g1i_h, g2i_h, ITEM_NUM)
        spmm(n1u_h, rows_h, cols_h, g1i_h, n2i_h, ITEM_NUM)

        r0b = s * bpt

        def plain_gather(idx_h, off, tabs, out_h):
            for h in range(bpt // SUB):
                pltpu.sync_copy(idx_h.at[pl.ds(off + h * SUB, SUB)],
                                gidx4.at[0])
                for kl in range(2):
                    ck = 2 * c + kl
                    for t, tab in enumerate(tabs):
                        pltpu.async_copy(tab.at[ck].at[gidx4.at[0]],
                                         gath3.at[0], semg).wait()
                        pltpu.sync_copy(
                            gath3.at[0],
                            out_h.at[pl.ds(r0b + h * SUB, SUB),
                                     t * NCHUNK + ck])

        def noisy_gather(idx_h, off, tabs, out_h):
            for h in range(bpt // SUB):
                pltpu.sync_copy(idx_h.at[pl.ds(off + h * SUB, SUB)],
                                gidx4.at[0])
                for kl in range(2):
                    ck = 2 * c + kl
                    pltpu.async_copy(noise_h.at[ck].at[gidx4.at[0]],
                                     gath3.at[1], semg).wait()
                    for t, tab in enumerate(tabs):
                        pltpu.async_copy(tab.at[ck].at[gidx4.at[0]],
                                         gath3.at[0], semg).wait()

                        def addrow(rr, carry):
                            gath3[0, rr, :] = gath3[0, rr, :] + gath3[1, rr, :]
                            return carry

                        lax.fori_loop(0, SUB, addrow, 0, unroll=False)
                        pltpu.sync_copy(
                            gath3.at[0],
                            out_h.at[pl.ds(r0b + h * SUB, SUB),
                                     t * NCHUNK + ck])

        plain_gather(ub_idx_h, r0b, (users_h, g1u_h, g2u_h), ub_h)
        base_tabs = (items_h, g1i_h, g2i_h)
        noisy_tabs = (items_h, g1i_h, n2i_h)

        @pl.when(s < half_tiles)
        def _():
            plain_gather(ib_idx_h, r0b, base_tabs, pt_h)
            noisy_gather(jb_idx_h, r0b + B // 2, noisy_tabs, nt_h)

        @pl.when(s >= half_tiles)
        def _():
            noisy_gather(ib_idx_h, r0b, noisy_tabs, pt_h)
            plain_gather(jb_idx_h, r0b - B // 2, base_tabs, nt_h)

    return k(users4, items4, noise4, rows_p, cols_p, vals_p,
             u_batch, i_batch, j_batch)


def _final_body(u_ref, p_ref, n_ref, o_ref):
    u = u_ref[...]
    o_ref[0, :] = jnp.sum(u * p_ref[...], axis=-1)
    o_ref[1, :] = jnp.sum(u * n_ref[...], axis=-1)


def _final_dots(ub, pt, nt):
    n = ub.shape[0]
    blk = 2048
    return pl.pallas_call(
        _final_body,
        out_shape=jax.ShapeDtypeStruct((2, n), jnp.float32),
        grid=(n // blk,),
        in_specs=[
            pl.BlockSpec((blk, ub.shape[1]), lambda i: (i, 0)),
            pl.BlockSpec((blk, ub.shape[1]), lambda i: (i, 0)),
            pl.BlockSpec((blk, ub.shape[1]), lambda i: (i, 0)),
        ],
        out_specs=pl.BlockSpec((2, blk), lambda i: (0, i)),
    )(ub, pt, nt)


def _chunked(x):
    n = x.shape[0]
    return x.reshape(n, NCHUNK, CW).transpose(1, 0, 2)


def kernel(u_batch, i_batch, j_batch, embed_user, embed_item, causal_user,
           causal_item, noise_item, ui_rows, ui_cols, ui_vals):
    users4 = _chunked(BPR_W * embed_user + CAUSAL_W * causal_user)
    items4 = _chunked(BPR_W * embed_item + CAUSAL_W * causal_item)
    noise4 = _chunked(noise_item)

    pad = NNZ_PAD - ui_rows.shape[0]
    rows_p = jnp.concatenate((ui_rows.astype(jnp.int32),
                              jnp.zeros((pad,), jnp.int32)))
    cols_p = jnp.concatenate((ui_cols.astype(jnp.int32),
                              jnp.zeros((pad,), jnp.int32)))
    vals_p = jnp.concatenate((ui_vals, jnp.zeros((pad,), jnp.float32)))

    B = u_batch.shape[0]
    outs = _gcn_sc(users4, items4, noise4, rows_p, cols_p, vals_p,
                   u_batch.astype(jnp.int32), i_batch.astype(jnp.int32),
                   j_batch.astype(jnp.int32))
    ub_h, pt_h, nt_h = outs[6], outs[7], outs[8]
    return _final_dots(ub_h.reshape(B, 3 * D), pt_h.reshape(B, 3 * D),
                       nt_h.reshape(B, 3 * D))


# speedup vs baseline: 2.5981x; 1.0283x over previous
import functools

import jax
import jax.numpy as jnp
from jax import lax
from jax.experimental import pallas as pl
from jax.experimental.pallas import tpu as pltpu
from jax.experimental.pallas import tpu_sc as plsc

USER_NUM = 100000
ITEM_NUM = 50000
D = 64
BPR_W = 0.7
CAUSAL_W = 0.3

NCHUNK = 4
CW = 16
NS = 16
E_BLK = 512
NB = 123
NNZ_PAD = NS * NB * E_BLK
GRPS = E_BLK // 16
SUB = 512


def _gcn_sc(users4, items4, noise4, rows_p, cols_p, vals_p,
            u_batch, i_batch, j_batch):
    f32 = jnp.float32
    B = u_batch.shape[0]
    bpt = B // NS
    half_tiles = (B // 2) // bpt
    u_sds = jax.ShapeDtypeStruct((NCHUNK, USER_NUM, CW), f32)
    i_sds = jax.ShapeDtypeStruct((NCHUNK, ITEM_NUM, CW), f32)
    b_sds = jax.ShapeDtypeStruct((B, 3 * NCHUNK, CW), f32)
    mesh = plsc.VectorSubcoreMesh(core_axis_name="c", subcore_axis_name="s")

    @functools.partial(
        pl.kernel,
        out_type=(u_sds, u_sds, i_sds, i_sds, u_sds, i_sds,
                  b_sds, b_sds, b_sds),
        mesh=mesh,
        compiler_params=pltpu.CompilerParams(use_tc_tiling_on_sc=False),
        scratch_types=[
            pltpu.VMEM_SHARED((USER_NUM, CW), f32),
            pltpu.VMEM((3, E_BLK, CW), f32),
            pltpu.VMEM((4, E_BLK), jnp.int32),
            pltpu.VMEM((4, E_BLK), jnp.int32),
            pltpu.VMEM((4, E_BLK), f32),
            pltpu.SemaphoreType.DMA,
            pltpu.SemaphoreType.DMA,
            pltpu.SemaphoreType.DMA,
        ],
    )
    def k(users_h, items_h, noise_h, rows_h, cols_h, vals_h,
          ub_idx_h, ib_idx_h, jb_idx_h,
          g1u_h, g2u_h, g1i_h, g2i_h, n1u_h, n2i_h,
          ub_h, pt_h, nt_h,
          acc, gath3, gidx4, sidx4, valsv4, semi, semg, semsc):
        c = lax.axis_index("c")
        s = lax.axis_index("s")

        def spmm(src_h, gidx_h, sidx_h, init_h, out_h, n_out):
            rpt = n_out // NS
            r0 = s * rpt
            for kl in range(2):
                ck = 2 * c + kl
                pltpu.sync_copy(init_h.at[ck, pl.ds(r0, rpt)],
                                acc.at[pl.ds(r0, rpt)])
                plsc.subcore_barrier()

                def issue_idx(b, r):
                    e0 = (s * NB + b) * E_BLK
                    pltpu.async_copy(gidx_h.at[pl.ds(e0, E_BLK)],
                                     gidx4.at[r], semi)
                    pltpu.async_copy(sidx_h.at[pl.ds(e0, E_BLK)],
                                     sidx4.at[r], semi)
                    pltpu.async_copy(vals_h.at[pl.ds(e0, E_BLK)],
                                     valsv4.at[r], semi)

                def wait_idx(b, r):
                    e0 = (s * NB + b) * E_BLK
                    pltpu.make_async_copy(gidx_h.at[pl.ds(e0, E_BLK)],
                                          gidx4.at[r], semi).wait()
                    pltpu.make_async_copy(sidx_h.at[pl.ds(e0, E_BLK)],
                                          sidx4.at[r], semi).wait()
                    pltpu.make_async_copy(vals_h.at[pl.ds(e0, E_BLK)],
                                          valsv4.at[r], semi).wait()

                def issue_gather(r, p):
                    pltpu.async_copy(src_h.at[ck].at[gidx4.at[r]],
                                     gath3.at[p], semg)

                def wait_gather(r, p):
                    pltpu.make_async_copy(src_h.at[ck].at[gidx4.at[r]],
                                          gath3.at[p], semg).wait()

                def issue_scatter(r, p):
                    pltpu.async_copy(gath3.at[p], acc.at[sidx4.at[r]],
                                     semsc, add=True)

                def wait_scatter(r, p):
                    pltpu.make_async_copy(gath3.at[p], acc.at[sidx4.at[r]],
                                          semsc).wait()

                def scale(r, p):
                    @plsc.parallel_loop(0, GRPS, step=1, unroll=2)
                    def grp(g):
                        vv = valsv4[r, pl.ds(g * 16, 16)]
                        for j in range(16):
                            e = g * 16 + j
                            gath3[p, e, :] = gath3[p, e, :] * vv[j]

                issue_idx(0, 0)

                def blk(b, carry):
                    p = lax.rem(b, 3)
                    r = lax.rem(b, 4)

                    @pl.when(b >= 3)
                    def _():
                        wait_scatter(lax.rem(b + 1, 4), lax.rem(b, 3))

                    @pl.when(b + 1 < NB)
                    def _():
                        issue_idx(b + 1, lax.rem(b + 1, 4))

                    wait_idx(b, r)
                    issue_gather(r, p)

                    @pl.when(b >= 1)
                    def _():
                        rm1 = lax.rem(b + 3, 4)
                        pm1 = lax.rem(b + 2, 3)
                        wait_gather(rm1, pm1)
                        scale(rm1, pm1)
                        issue_scatter(rm1, pm1)

                    return carry

                lax.fori_loop(0, NB, blk, 0, unroll=False)
                rL, pL = (NB - 1) % 4, (NB - 1) % 3
                wait_gather(rL, pL)
                scale(rL, pL)
                issue_scatter(rL, pL)
                wait_scatter((NB - 3) % 4, (NB - 3) % 3)
                wait_scatter((NB - 2) % 4, (NB - 2) % 3)
                wait_scatter(rL, pL)
                plsc.subcore_barrier()
                pltpu.sync_copy(acc.at[pl.ds(r0, rpt)],
                                out_h.at[ck, pl.ds(r0, rpt)])
                plsc.subcore_barrier()

        spmm(users_h, rows_h, cols_h, items_h, g1i_h, ITEM_NUM)
        spmm(items_h, cols_h, rows_h, users_h, g1u_h, USER_NUM)
        spmm(noise_h, cols_h, rows_h, g1u_h, n1u_h, USER_NUM)
        spmm(g1i_h, cols_h, rows_h, g1u_h, g2u_h, USER_NUM)
        spmm(g1u_h, rows_h, cols_h, g1i_h, g2i_h, ITEM_NUM)
        spmm(n1u_h, rows_h, cols_h, g1i_h, n2i_h, ITEM_NUM)

        r0b = s * bpt

        def plain_gather(idx_h, off, tabs, out_h):
            for h in range(bpt // SUB):
                pltpu.sync_copy(idx_h.at[pl.ds(off + h * SUB, SUB)],
                                gidx4.at[0])
                for kl in range(2):
                    ck = 2 * c + kl
                    for t, tab in enumerate(tabs):
                        pltpu.async_copy(tab.at[ck].at[gidx4.at[0]],
                                         gath3.at[0], semg).wait()
                        pltpu.sync_copy(
                            gath3.at[0],
                            out_h.at[pl.ds(r0b + h * SUB, SUB),
                                     t * NCHUNK + ck])

        def noisy_gather(idx_h, off, tabs, out_h):
            for h in range(bpt // SUB):
                pltpu.sync_copy(idx_h.at[pl.ds(off + h * SUB, SUB)],
                                gidx4.at[0])
                for kl in range(2):
                    ck = 2 * c + kl
                    pltpu.async_copy(noise_h.at[ck].at[gidx4.at[0]],
                                     gath3.at[1], semg).wait()
                    for t, tab in enumerate(tabs):
                        pltpu.async_copy(tab.at[ck].at[gidx4.at[0]],
                                         gath3.at[0], semg).wait()

                        def addrow(rr, carry):
                            gath3[0, rr, :] = gath3[0, rr, :] + gath3[1, rr, :]
                            return carry

                        lax.fori_loop(0, SUB, addrow, 0, unroll=False)
                        pltpu.sync_copy(
                            gath3.at[0],
                            out_h.at[pl.ds(r0b + h * SUB, SUB),
                                     t * NCHUNK + ck])

        plain_gather(ub_idx_h, r0b, (users_h, g1u_h, g2u_h), ub_h)
        base_tabs = (items_h, g1i_h, g2i_h)
        noisy_tabs = (items_h, g1i_h, n2i_h)

        @pl.when(s < half_tiles)
        def _():
            plain_gather(ib_idx_h, r0b, base_tabs, pt_h)
            noisy_gather(jb_idx_h, r0b + B // 2, noisy_tabs, nt_h)

        @pl.when(s >= half_tiles)
        def _():
            noisy_gather(ib_idx_h, r0b, noisy_tabs, pt_h)
            plain_gather(jb_idx_h, r0b - B // 2, base_tabs, nt_h)

    return k(users4, items4, noise4, rows_p, cols_p, vals_p,
             u_batch, i_batch, j_batch)


def _final_body(u_ref, p_ref, n_ref, o_ref):
    u = u_ref[...]
    o_ref[0, :] = jnp.sum(u * p_ref[...], axis=-1)
    o_ref[1, :] = jnp.sum(u * n_ref[...], axis=-1)


def _final_dots(ub, pt, nt):
    n = ub.shape[0]
    blk = 2048
    return pl.pallas_call(
        _final_body,
        out_shape=jax.ShapeDtypeStruct((2, n), jnp.float32),
        grid=(n // blk,),
        in_specs=[
            pl.BlockSpec((blk, ub.shape[1]), lambda i: (i, 0)),
            pl.BlockSpec((blk, ub.shape[1]), lambda i: (i, 0)),
            pl.BlockSpec((blk, ub.shape[1]), lambda i: (i, 0)),
        ],
        out_specs=pl.BlockSpec((2, blk), lambda i: (0, i)),
    )(ub, pt, nt)


def _chunked(x):
    n = x.shape[0]
    return x.reshape(n, NCHUNK, CW).transpose(1, 0, 2)


def kernel(u_batch, i_batch, j_batch, embed_user, embed_item, causal_user,
           causal_item, noise_item, ui_rows, ui_cols, ui_vals):
    users4 = _chunked(BPR_W * embed_user + CAUSAL_W * causal_user)
    items4 = _chunked(BPR_W * embed_item + CAUSAL_W * causal_item)
    noise4 = _chunked(noise_item)

    pad = NNZ_PAD - ui_rows.shape[0]
    rows_p = jnp.concatenate((ui_rows.astype(jnp.int32),
                              jnp.zeros((pad,), jnp.int32)))
    cols_p = jnp.concatenate((ui_cols.astype(jnp.int32),
                              jnp.zeros((pad,), jnp.int32)))
    vals_p = jnp.concatenate((ui_vals, jnp.zeros((pad,), jnp.float32)))

    B = u_batch.shape[0]
    outs = _gcn_sc(users4, items4, noise4, rows_p, cols_p, vals_p,
                   u_batch.astype(jnp.int32), i_batch.astype(jnp.int32),
                   j_batch.astype(jnp.int32))
    ub_h, pt_h, nt_h = outs[6], outs[7], outs[8]
    return _final_dots(ub_h.reshape(B, 3 * D), pt_h.reshape(B, 3 * D),
                       nt_h.reshape(B, 3 * D))


# speedup vs baseline: 2.6190x; 1.0080x over previous
import functools

import jax
import jax.numpy as jnp
from jax import lax
from jax.experimental import pallas as pl
from jax.experimental.pallas import tpu as pltpu
from jax.experimental.pallas import tpu_sc as plsc

USER_NUM = 100000
ITEM_NUM = 50000
D = 64
BPR_W = 0.7
CAUSAL_W = 0.3

NCHUNK = 4
CW = 16
NS = 16
E_BLK = 512
NB = 123
NNZ_PAD = NS * NB * E_BLK
GRPS = E_BLK // 16
SUB = 512


def _gcn_sc(users4, items4, noise4, rows_p, cols_p, vals_p,
            u_batch, i_batch, j_batch):
    f32 = jnp.float32
    B = u_batch.shape[0]
    bpt = B // NS
    half_tiles = (B // 2) // bpt
    u_sds = jax.ShapeDtypeStruct((NCHUNK, USER_NUM, CW), f32)
    i_sds = jax.ShapeDtypeStruct((NCHUNK, ITEM_NUM, CW), f32)
    b_sds = jax.ShapeDtypeStruct((B, 3 * NCHUNK, CW), f32)
    mesh = plsc.VectorSubcoreMesh(core_axis_name="c", subcore_axis_name="s")

    @functools.partial(
        pl.kernel,
        out_type=(u_sds, u_sds, i_sds, i_sds, u_sds, i_sds,
                  b_sds, b_sds, b_sds),
        mesh=mesh,
        compiler_params=pltpu.CompilerParams(use_tc_tiling_on_sc=False),
        scratch_types=[
            pltpu.VMEM_SHARED((USER_NUM, CW), f32),
            pltpu.VMEM((3, E_BLK, CW), f32),
            pltpu.VMEM((4, E_BLK), jnp.int32),
            pltpu.VMEM((4, E_BLK), jnp.int32),
            pltpu.VMEM((4, E_BLK), f32),
            pltpu.SemaphoreType.DMA,
            pltpu.SemaphoreType.DMA,
            pltpu.SemaphoreType.DMA,
        ],
    )
    def k(users_h, items_h, noise_h, rows_h, cols_h, vals_h,
          ub_idx_h, ib_idx_h, jb_idx_h,
          g1u_h, g2u_h, g1i_h, g2i_h, n1u_h, n2i_h,
          ub_h, pt_h, nt_h,
          acc, gath3, gidx4, sidx4, valsv4, semi, semg, semsc):
        c = lax.axis_index("c")
        s = lax.axis_index("s")

        def spmm(src_h, gidx_h, sidx_h, init_h, out_h, n_out):
            rpt = n_out // NS
            r0 = s * rpt
            for kl in range(2):
                ck = 2 * c + kl
                pltpu.sync_copy(init_h.at[ck, pl.ds(r0, rpt)],
                                acc.at[pl.ds(r0, rpt)])
                plsc.subcore_barrier()

                def issue_idx(b, r):
                    e0 = (s * NB + b) * E_BLK
                    pltpu.async_copy(gidx_h.at[pl.ds(e0, E_BLK)],
                                     gidx4.at[r], semi)
                    pltpu.async_copy(sidx_h.at[pl.ds(e0, E_BLK)],
                                     sidx4.at[r], semi)
                    pltpu.async_copy(vals_h.at[pl.ds(e0, E_BLK)],
                                     valsv4.at[r], semi)

                def wait_gidx(b, r):
                    e0 = (s * NB + b) * E_BLK
                    pltpu.make_async_copy(gidx_h.at[pl.ds(e0, E_BLK)],
                                          gidx4.at[r], semi).wait()

                def wait_sv(b, r):
                    e0 = (s * NB + b) * E_BLK
                    pltpu.make_async_copy(sidx_h.at[pl.ds(e0, E_BLK)],
                                          sidx4.at[r], semi).wait()
                    pltpu.make_async_copy(vals_h.at[pl.ds(e0, E_BLK)],
                                          valsv4.at[r], semi).wait()

                def issue_gather(r, p):
                    pltpu.async_copy(src_h.at[ck].at[gidx4.at[r]],
                                     gath3.at[p], semg)

                def wait_gather(r, p):
                    pltpu.make_async_copy(src_h.at[ck].at[gidx4.at[r]],
                                          gath3.at[p], semg).wait()

                def issue_scatter(r, p):
                    pltpu.async_copy(gath3.at[p], acc.at[sidx4.at[r]],
                                     semsc, add=True)

                def wait_scatter(r, p):
                    pltpu.make_async_copy(gath3.at[p], acc.at[sidx4.at[r]],
                                          semsc).wait()

                def scale(r, p):
                    @plsc.parallel_loop(0, GRPS, step=1, unroll=2)
                    def grp(g):
                        vv = valsv4[r, pl.ds(g * 16, 16)]
                        for j in range(16):
                            e = g * 16 + j
                            gath3[p, e, :] = gath3[p, e, :] * vv[j]

                issue_idx(0, 0)

                def blk(b, carry):
                    p = lax.rem(b, 3)
                    r = lax.rem(b, 4)

                    @pl.when(b >= 3)
                    def _():
                        wait_scatter(lax.rem(b + 1, 4), lax.rem(b, 3))

                    @pl.when(b + 1 < NB)
                    def _():
                        issue_idx(b + 1, lax.rem(b + 1, 4))

                    wait_gidx(b, r)
                    issue_gather(r, p)

                    @pl.when(b >= 1)
                    def _():
                        rm1 = lax.rem(b + 3, 4)
                        pm1 = lax.rem(b + 2, 3)
                        wait_sv(b - 1, rm1)
                        wait_gather(rm1, pm1)
                        scale(rm1, pm1)
                        issue_scatter(rm1, pm1)

                    return carry

                lax.fori_loop(0, NB, blk, 0, unroll=False)
                rL, pL = (NB - 1) % 4, (NB - 1) % 3
                wait_sv(NB - 1, rL)
                wait_gather(rL, pL)
                scale(rL, pL)
                issue_scatter(rL, pL)
                wait_scatter((NB - 3) % 4, (NB - 3) % 3)
                wait_scatter((NB - 2) % 4, (NB - 2) % 3)
                wait_scatter(rL, pL)
                plsc.subcore_barrier()
                pltpu.sync_copy(acc.at[pl.ds(r0, rpt)],
                                out_h.at[ck, pl.ds(r0, rpt)])
                plsc.subcore_barrier()

        spmm(users_h, rows_h, cols_h, items_h, g1i_h, ITEM_NUM)
        spmm(items_h, cols_h, rows_h, users_h, g1u_h, USER_NUM)
        spmm(noise_h, cols_h, rows_h, g1u_h, n1u_h, USER_NUM)
        spmm(g1i_h, cols_h, rows_h, g1u_h, g2u_h, USER_NUM)
        spmm(g1u_h, rows_h, cols_h, g1i_h, g2i_h, ITEM_NUM)
        spmm(n1u_h, rows_h, cols_h, g1i_h, n2i_h, ITEM_NUM)

        r0b = s * bpt

        def plain_gather(idx_h, off, tabs, out_h):
            for h in range(bpt // SUB):
                pltpu.sync_copy(idx_h.at[pl.ds(off + h * SUB, SUB)],
                                gidx4.at[0])
                for kl in range(2):
                    ck = 2 * c + kl
                    for t, tab in enumerate(tabs):
                        pltpu.async_copy(tab.at[ck].at[gidx4.at[0]],
                                         gath3.at[0], semg).wait()
                        pltpu.sync_copy(
                            gath3.at[0],
                            out_h.at[pl.ds(r0b + h * SUB, SUB),
                                     t * NCHUNK + ck])

        def noisy_gather(idx_h, off, tabs, out_h):
            for h in range(bpt // SUB):
                pltpu.sync_copy(idx_h.at[pl.ds(off + h * SUB, SUB)],
                                gidx4.at[0])
                for kl in range(2):
                    ck = 2 * c + kl
                    pltpu.async_copy(noise_h.at[ck].at[gidx4.at[0]],
                                     gath3.at[1], semg).wait()
                    for t, tab in enumerate(tabs):
                        pltpu.async_copy(tab.at[ck].at[gidx4.at[0]],
                                         gath3.at[0], semg).wait()

                        def addrow(rr, carry):
                            gath3[0, rr, :] = gath3[0, rr, :] + gath3[1, rr, :]
                            return carry

                        lax.fori_loop(0, SUB, addrow, 0, unroll=False)
                        pltpu.sync_copy(
                            gath3.at[0],
                            out_h.at[pl.ds(r0b + h * SUB, SUB),
                                     t * NCHUNK + ck])

        plain_gather(ub_idx_h, r0b, (users_h, g1u_h, g2u_h), ub_h)
        base_tabs = (items_h, g1i_h, g2i_h)
        noisy_tabs = (items_h, g1i_h, n2i_h)

        @pl.when(s < half_tiles)
        def _():
            plain_gather(ib_idx_h, r0b, base_tabs, pt_h)
            noisy_gather(jb_idx_h, r0b + B // 2, noisy_tabs, nt_h)

        @pl.when(s >= half_tiles)
        def _():
            noisy_gather(ib_idx_h, r0b, noisy_tabs, pt_h)
            plain_gather(jb_idx_h, r0b - B // 2, base_tabs, nt_h)

    return k(users4, items4, noise4, rows_p, cols_p, vals_p,
             u_batch, i_batch, j_batch)


def _final_body(u_ref, p_ref, n_ref, o_ref):
    u = u_ref[...]
    o_ref[0, :] = jnp.sum(u * p_ref[...], axis=-1)
    o_ref[1, :] = jnp.sum(u * n_ref[...], axis=-1)


def _final_dots(ub, pt, nt):
    n = ub.shape[0]
    blk = 2048
    return pl.pallas_call(
        _final_body,
        out_shape=jax.ShapeDtypeStruct((2, n), jnp.float32),
        grid=(n // blk,),
        in_specs=[
            pl.BlockSpec((blk, ub.shape[1]), lambda i: (i, 0)),
            pl.BlockSpec((blk, ub.shape[1]), lambda i: (i, 0)),
            pl.BlockSpec((blk, ub.shape[1]), lambda i: (i, 0)),
        ],
        out_specs=pl.BlockSpec((2, blk), lambda i: (0, i)),
    )(ub, pt, nt)


def _chunked(x):
    n = x.shape[0]
    return x.reshape(n, NCHUNK, CW).transpose(1, 0, 2)


def kernel(u_batch, i_batch, j_batch, embed_user, embed_item, causal_user,
           causal_item, noise_item, ui_rows, ui_cols, ui_vals):
    users4 = _chunked(BPR_W * embed_user + CAUSAL_W * causal_user)
    items4 = _chunked(BPR_W * embed_item + CAUSAL_W * causal_item)
    noise4 = _chunked(noise_item)

    pad = NNZ_PAD - ui_rows.shape[0]
    rows_p = jnp.concatenate((ui_rows.astype(jnp.int32),
                              jnp.zeros((pad,), jnp.int32)))
    cols_p = jnp.concatenate((ui_cols.astype(jnp.int32),
                              jnp.zeros((pad,), jnp.int32)))
    vals_p = jnp.concatenate((ui_vals, jnp.zeros((pad,), jnp.float32)))

    B = u_batch.shape[0]
    outs = _gcn_sc(users4, items4, noise4, rows_p, cols_p, vals_p,
                   u_batch.astype(jnp.int32), i_batch.astype(jnp.int32),
                   j_batch.astype(jnp.int32))
    ub_h, pt_h, nt_h = outs[6], outs[7], outs[8]
    return _final_dots(ub_h.reshape(B, 3 * D), pt_h.reshape(B, 3 * D),
                       nt_h.reshape(B, 3 * D))


# speedup vs baseline: 2.7851x; 1.0634x over previous
import functools

import jax
import jax.numpy as jnp
from jax import lax
from jax.experimental import pallas as pl
from jax.experimental.pallas import tpu as pltpu
from jax.experimental.pallas import tpu_sc as plsc

USER_NUM = 100000
ITEM_NUM = 50000
D = 64
BPR_W = 0.7
CAUSAL_W = 0.3

NCHUNK = 4
CW = 16
NS = 16
E_BLK = 512
NB = 123
NNZ_PAD = NS * NB * E_BLK
GRPS = E_BLK // 16
SUB = 512


def _gcn_sc(users4, items4, noise4, rows_p, cols_p, vals_p,
            u_batch, i_batch, j_batch):
    f32 = jnp.float32
    B = u_batch.shape[0]
    bpt = B // NS
    half_tiles = (B // 2) // bpt
    u_sds = jax.ShapeDtypeStruct((NCHUNK, USER_NUM, CW), f32)
    i_sds = jax.ShapeDtypeStruct((NCHUNK, ITEM_NUM, CW), f32)
    b_sds = jax.ShapeDtypeStruct((B, 3 * NCHUNK, CW), f32)
    mesh = plsc.VectorSubcoreMesh(core_axis_name="c", subcore_axis_name="s")

    @functools.partial(
        pl.kernel,
        out_type=(u_sds, u_sds, i_sds, i_sds, u_sds, i_sds,
                  b_sds, b_sds, b_sds),
        mesh=mesh,
        compiler_params=pltpu.CompilerParams(use_tc_tiling_on_sc=False),
        scratch_types=[
            pltpu.VMEM_SHARED((USER_NUM, CW), f32),
            pltpu.VMEM((3, E_BLK, CW), f32),
            pltpu.VMEM((4, E_BLK), jnp.int32),
            pltpu.VMEM((4, E_BLK), jnp.int32),
            pltpu.VMEM((4, E_BLK), f32),
            pltpu.SemaphoreType.DMA((4,)),
            pltpu.SemaphoreType.DMA((3,)),
            pltpu.SemaphoreType.DMA((3,)),
        ],
    )
    def k(users_h, items_h, noise_h, rows_h, cols_h, vals_h,
          ub_idx_h, ib_idx_h, jb_idx_h,
          g1u_h, g2u_h, g1i_h, g2i_h, n1u_h, n2i_h,
          ub_h, pt_h, nt_h,
          acc, gath3, gidx4, sidx4, valsv4, semi, semg, semsc):
        c = lax.axis_index("c")
        s = lax.axis_index("s")

        def spmm(src_h, gidx_h, sidx_h, init_h, out_h, n_out):
            rpt = n_out // NS
            r0 = s * rpt
            for kl in range(2):
                ck = 2 * c + kl
                pltpu.sync_copy(init_h.at[ck, pl.ds(r0, rpt)],
                                acc.at[pl.ds(r0, rpt)])
                plsc.subcore_barrier()

                def issue_idx(b, r):
                    e0 = (s * NB + b) * E_BLK
                    pltpu.async_copy(gidx_h.at[pl.ds(e0, E_BLK)],
                                     gidx4.at[r], semi.at[r])
                    pltpu.async_copy(sidx_h.at[pl.ds(e0, E_BLK)],
                                     sidx4.at[r], semi.at[r])
                    pltpu.async_copy(vals_h.at[pl.ds(e0, E_BLK)],
                                     valsv4.at[r], semi.at[r])

                def wait_idx(b, r):
                    e0 = (s * NB + b) * E_BLK
                    pltpu.make_async_copy(gidx_h.at[pl.ds(e0, E_BLK)],
                                          gidx4.at[r], semi.at[r]).wait()
                    pltpu.make_async_copy(sidx_h.at[pl.ds(e0, E_BLK)],
                                          sidx4.at[r], semi.at[r]).wait()
                    pltpu.make_async_copy(vals_h.at[pl.ds(e0, E_BLK)],
                                          valsv4.at[r], semi.at[r]).wait()

                def issue_gather(r, p):
                    pltpu.async_copy(src_h.at[ck].at[gidx4.at[r]],
                                     gath3.at[p], semg.at[p])

                def wait_gather(r, p):
                    pltpu.make_async_copy(src_h.at[ck].at[gidx4.at[r]],
                                          gath3.at[p], semg.at[p]).wait()

                def issue_scatter(r, p):
                    pltpu.async_copy(gath3.at[p], acc.at[sidx4.at[r]],
                                     semsc.at[p], add=True)

                def wait_scatter(r, p):
                    pltpu.make_async_copy(gath3.at[p], acc.at[sidx4.at[r]],
                                          semsc.at[p]).wait()

                def scale(r, p):
                    @plsc.parallel_loop(0, GRPS, step=1, unroll=2)
                    def grp(g):
                        vv = valsv4[r, pl.ds(g * 16, 16)]
                        for j in range(16):
                            e = g * 16 + j
                            gath3[p, e, :] = gath3[p, e, :] * vv[j]

                issue_idx(0, 0)

                def blk(b, carry):
                    p = lax.rem(b, 3)
                    r = lax.rem(b, 4)

                    @pl.when(b >= 3)
                    def _():
                        wait_scatter(lax.rem(b + 1, 4), lax.rem(b, 3))

                    @pl.when(b + 1 < NB)
                    def _():
                        issue_idx(b + 1, lax.rem(b + 1, 4))

                    wait_idx(b, r)
                    issue_gather(r, p)

                    @pl.when(b >= 1)
                    def _():
                        rm1 = lax.rem(b + 3, 4)
                        pm1 = lax.rem(b + 2, 3)
                        wait_gather(rm1, pm1)
                        scale(rm1, pm1)
                        issue_scatter(rm1, pm1)

                    return carry

                lax.fori_loop(0, NB, blk, 0, unroll=False)
                rL, pL = (NB - 1) % 4, (NB - 1) % 3
                wait_gather(rL, pL)
                scale(rL, pL)
                issue_scatter(rL, pL)
                wait_scatter((NB - 3) % 4, (NB - 3) % 3)
                wait_scatter((NB - 2) % 4, (NB - 2) % 3)
                wait_scatter(rL, pL)
                plsc.subcore_barrier()
                pltpu.sync_copy(acc.at[pl.ds(r0, rpt)],
                                out_h.at[ck, pl.ds(r0, rpt)])
                plsc.subcore_barrier()

        spmm(users_h, rows_h, cols_h, items_h, g1i_h, ITEM_NUM)
        spmm(items_h, cols_h, rows_h, users_h, g1u_h, USER_NUM)
        spmm(noise_h, cols_h, rows_h, g1u_h, n1u_h, USER_NUM)
        spmm(g1i_h, cols_h, rows_h, g1u_h, g2u_h, USER_NUM)
        spmm(g1u_h, rows_h, cols_h, g1i_h, g2i_h, ITEM_NUM)
        spmm(n1u_h, rows_h, cols_h, g1i_h, n2i_h, ITEM_NUM)

        r0b = s * bpt

        def plain_gather(idx_h, off, tabs, out_h):
            for h in range(bpt // SUB):
                pltpu.sync_copy(idx_h.at[pl.ds(off + h * SUB, SUB)],
                                gidx4.at[0])
                for kl in range(2):
                    ck = 2 * c + kl
                    for t, tab in enumerate(tabs):
                        pltpu.async_copy(tab.at[ck].at[gidx4.at[0]],
                                         gath3.at[0], semg.at[0]).wait()
                        pltpu.sync_copy(
                            gath3.at[0],
                            out_h.at[pl.ds(r0b + h * SUB, SUB),
                                     t * NCHUNK + ck])

        def noisy_gather(idx_h, off, tabs, out_h):
            for h in range(bpt // SUB):
                pltpu.sync_copy(idx_h.at[pl.ds(off + h * SUB, SUB)],
                                gidx4.at[0])
                for kl in range(2):
                    ck = 2 * c + kl
                    pltpu.async_copy(noise_h.at[ck].at[gidx4.at[0]],
                                     gath3.at[1], semg.at[1]).wait()
                    for t, tab in enumerate(tabs):
                        pltpu.async_copy(tab.at[ck].at[gidx4.at[0]],
                                         gath3.at[0], semg.at[0]).wait()

                        def addrow(rr, carry):
                            gath3[0, rr, :] = gath3[0, rr, :] + gath3[1, rr, :]
                            return carry

                        lax.fori_loop(0, SUB, addrow, 0, unroll=False)
                        pltpu.sync_copy(
                            gath3.at[0],
                            out_h.at[pl.ds(r0b + h * SUB, SUB),
                                     t * NCHUNK + ck])

        plain_gather(ub_idx_h, r0b, (users_h, g1u_h, g2u_h), ub_h)
        base_tabs = (items_h, g1i_h, g2i_h)
        noisy_tabs = (items_h, g1i_h, n2i_h)

        @pl.when(s < half_tiles)
        def _():
            plain_gather(ib_idx_h, r0b, base_tabs, pt_h)
            noisy_gather(jb_idx_h, r0b + B // 2, noisy_tabs, nt_h)

        @pl.when(s >= half_tiles)
        def _():
            noisy_gather(ib_idx_h, r0b, noisy_tabs, pt_h)
            plain_gather(jb_idx_h, r0b - B // 2, base_tabs, nt_h)

    return k(users4, items4, noise4, rows_p, cols_p, vals_p,
             u_batch, i_batch, j_batch)


def _final_body(u_ref, p_ref, n_ref, o_ref):
    u = u_ref[...]
    o_ref[0, :] = jnp.sum(u * p_ref[...], axis=-1)
    o_ref[1, :] = jnp.sum(u * n_ref[...], axis=-1)


def _final_dots(ub, pt, nt):
    n = ub.shape[0]
    blk = 2048
    return pl.pallas_call(
        _final_body,
        out_shape=jax.ShapeDtypeStruct((2, n), jnp.float32),
        grid=(n // blk,),
        in_specs=[
            pl.BlockSpec((blk, ub.shape[1]), lambda i: (i, 0)),
            pl.BlockSpec((blk, ub.shape[1]), lambda i: (i, 0)),
            pl.BlockSpec((blk, ub.shape[1]), lambda i: (i, 0)),
        ],
        out_specs=pl.BlockSpec((2, blk), lambda i: (0, i)),
    )(ub, pt, nt)


def _chunked(x):
    n = x.shape[0]
    return x.reshape(n, NCHUNK, CW).transpose(1, 0, 2)


def kernel(u_batch, i_batch, j_batch, embed_user, embed_item, causal_user,
           causal_item, noise_item, ui_rows, ui_cols, ui_vals):
    users4 = _chunked(BPR_W * embed_user + CAUSAL_W * causal_user)
    items4 = _chunked(BPR_W * embed_item + CAUSAL_W * causal_item)
    noise4 = _chunked(noise_item)

    pad = NNZ_PAD - ui_rows.shape[0]
    rows_p = jnp.concatenate((ui_rows.astype(jnp.int32),
                              jnp.zeros((pad,), jnp.int32)))
    cols_p = jnp.concatenate((ui_cols.astype(jnp.int32),
                              jnp.zeros((pad,), jnp.int32)))
    vals_p = jnp.concatenate((ui_vals, jnp.zeros((pad,), jnp.float32)))

    B = u_batch.shape[0]
    outs = _gcn_sc(users4, items4, noise4, rows_p, cols_p, vals_p,
                   u_batch.astype(jnp.int32), i_batch.astype(jnp.int32),
                   j_batch.astype(jnp.int32))
    ub_h, pt_h, nt_h = outs[6], outs[7], outs[8]
    return _final_dots(ub_h.reshape(B, 3 * D), pt_h.reshape(B, 3 * D),
                       nt_h.reshape(B, 3 * D))
